# Initial kernel scaffold; baseline (speedup 1.0000x reference)
import functools
import math

import jax
import jax.numpy as jnp
from jax import lax
from jax.experimental import pallas as pl
from jax.experimental.pallas import tpu as pltpu
from jax.experimental.pallas import tpu_sc as plsc

_RES_MIN = 16
_RES_MAX = 256
_L = 16
_T = 65536
_B = 4
_N = _RES_MAX * _RES_MAX
_NW = 32
_PAIRS = _B * _L
_PAIRS_PER_W = _PAIRS // _NW
_C = 8192
_CHUNKS = _N // _C
_HASH_K = jnp.int32(-1640531535)
_IDX_MASK = jnp.int32(_T - 1)


def _sc_body(tab_hbm, cx_hbm, cy_hbm, res_hbm, feats_hbm,
             tab_v, cx_v, cy_v, f0_v, f1_v, res_v):
    wid = lax.axis_index("s") * 2 + lax.axis_index("c")
    pltpu.sync_copy(res_hbm, res_v)
    resv = res_v[...]
    iot = lax.broadcasted_iota(jnp.int32, (16,), 0)

    for j in range(_PAIRS_PER_W):
        pair = wid * _PAIRS_PER_W + j
        b = pair // _L
        lvl = pair % _L
        r = jnp.sum(jnp.where(iot == lvl, resv, jnp.float32(0.0)))
        pltpu.sync_copy(tab_hbm.at[pl.ds(pair * _T, _T)], tab_v)
        for c in range(_CHUNKS):
            off = b * _N + c * _C
            pltpu.sync_copy(cx_hbm.at[pl.ds(off, _C)], cx_v)
            pltpu.sync_copy(cy_hbm.at[pl.ds(off, _C)], cy_v)

            def body(i, carry):
                sl = pl.ds(i * 16, 16)
                sx = cx_v[sl] * r
                sy = cy_v[sl] * r
                xi = sx.astype(jnp.int32)
                yi = sy.astype(jnp.int32)
                fx = sx - xi.astype(jnp.float32)
                fy = sy - yi.astype(jnp.float32)
                yk0 = yi * _HASH_K
                yk1 = yk0 + _HASH_K
                x1 = xi + 1
                i00 = (xi ^ yk0) & _IDX_MASK
                i10 = (x1 ^ yk0) & _IDX_MASK
                i01 = (xi ^ yk1) & _IDX_MASK
                i11 = (x1 ^ yk1) & _IDX_MASK
                g00 = plsc.load_gather(tab_v, [i00])
                g10 = plsc.load_gather(tab_v, [i10])
                g01 = plsc.load_gather(tab_v, [i01])
                g11 = plsc.load_gather(tab_v, [i11])

                def lo(g):
                    return lax.bitcast_convert_type(
                        jnp.left_shift(g, 16), jnp.float32)

                def hi(g):
                    return lax.bitcast_convert_type(
                        g & jnp.int32(-65536), jnp.float32)

                gx = 1.0 - fx
                gy = 1.0 - fy
                w00 = gx * gy
                w10 = fx * gy
                w01 = gx * fy
                w11 = fx * fy
                f0_v[sl] = (w00 * lo(g00) + w10 * lo(g10)
                            + w01 * lo(g01) + w11 * lo(g11))
                f1_v[sl] = (w00 * hi(g00) + w10 * hi(g10)
                            + w01 * hi(g01) + w11 * hi(g11))
                return carry

            lax.fori_loop(0, _C // 16, body, 0)
            fbase = pair * (2 * _N) + c * _C
            pltpu.sync_copy(f0_v, feats_hbm.at[pl.ds(fbase, _C)])
            pltpu.sync_copy(f1_v, feats_hbm.at[pl.ds(fbase + _N, _C)])


_sc_retrieve = functools.partial(
    pl.kernel,
    mesh=plsc.VectorSubcoreMesh(core_axis_name="c", subcore_axis_name="s"),
    out_type=jax.ShapeDtypeStruct((_PAIRS * 2 * _N,), jnp.float32),
    scratch_types=[
        pltpu.VMEM((_T,), jnp.int32),
        pltpu.VMEM((_C,), jnp.float32),
        pltpu.VMEM((_C,), jnp.float32),
        pltpu.VMEM((_C,), jnp.float32),
        pltpu.VMEM((_C,), jnp.float32),
        pltpu.VMEM((16,), jnp.float32),
    ],
)(_sc_body)


_BN = 8192


def _tc_body(s_ref, awt_ref, ab_ref, w8_ref, b8_ref, feats_ref, out_ref):
    style = jnp.dot(s_ref[...], awt_ref[...],
                    preferred_element_type=jnp.float32) + ab_ref[...]
    w = w8_ref[...] * style
    demod = lax.rsqrt(jnp.sum(w * w, axis=1, keepdims=True) + 1e-8)
    wd = w * demod
    out_ref[0] = jnp.dot(wd, feats_ref[0],
                         preferred_element_type=jnp.float32) + b8_ref[...]


def kernel(x, coords, s, weight, bias, affine_W, affine_b):
    b = x.shape[0]
    tabs = x.reshape(b, _L, _T, 2).astype(jnp.bfloat16)
    tu = lax.bitcast_convert_type(tabs, jnp.uint16).astype(jnp.uint32)
    packed = lax.bitcast_convert_type(
        tu[..., 0] | (tu[..., 1] << 16), jnp.int32).reshape(-1)
    cx = coords[..., 0].reshape(-1)
    cy = coords[..., 1].reshape(-1)
    growth = math.exp((math.log(_RES_MAX) - math.log(_RES_MIN)) / (_L - 1))
    res = jnp.floor(_RES_MIN * growth ** jnp.arange(_L, dtype=jnp.float32))

    feats = _sc_retrieve(packed, cx, cy, res).reshape(b, 2 * _L, _N)

    w8 = jnp.zeros((8, 2 * _L), jnp.float32).at[:3].set(weight)
    b8 = jnp.zeros((8, 1), jnp.float32).at[:3, 0].set(bias)
    out_pad = pl.pallas_call(
        _tc_body,
        grid=(b, _N // _BN),
        in_specs=[
            pl.BlockSpec((1, 512), lambda i, n: (i, 0)),
            pl.BlockSpec((512, 2 * _L), lambda i, n: (0, 0)),
            pl.BlockSpec((1, 2 * _L), lambda i, n: (0, 0)),
            pl.BlockSpec((8, 2 * _L), lambda i, n: (0, 0)),
            pl.BlockSpec((8, 1), lambda i, n: (0, 0)),
            pl.BlockSpec((1, 2 * _L, _BN), lambda i, n: (i, 0, n)),
        ],
        out_specs=pl.BlockSpec((1, 8, _BN), lambda i, n: (i, 0, n)),
        out_shape=jax.ShapeDtypeStruct((b, 8, _N), jnp.float32),
    )(s, affine_W.T, affine_b.reshape(1, -1), w8, b8, feats)

    return out_pad[:, :3, :].reshape(b, 3, _RES_MAX, _RES_MAX)


# speedup vs baseline: 283.2411x; 283.2411x over previous
import functools
import math

import jax
import jax.numpy as jnp
import numpy as np
from jax import lax
from jax.experimental import pallas as pl
from jax.experimental.pallas import tpu as pltpu
from jax.experimental.pallas import tpu_sc as plsc

_RES_MIN = 16
_RES_MAX = 256
_L = 16
_T = 65536
_B = 4
_N = _RES_MAX * _RES_MAX
_NW = 32
_PAIRS = _B * _L
_PAIRS_PER_W = _PAIRS // _NW
_C = 8192
_CHUNKS = _N // _C
_HASH_K = np.int32(-1640531535)
_IDX_MASK = np.int32(_T - 1)


def _sc_body(tab_hbm, cx_hbm, cy_hbm, res_hbm, feats_hbm,
             tab_v, cx_v, cy_v, f0_v, f1_v, res_v):
    wid = lax.axis_index("s") * 2 + lax.axis_index("c")
    pltpu.sync_copy(res_hbm, res_v)

    for j in range(_PAIRS_PER_W):
        pair = wid * _PAIRS_PER_W + j
        b = pair // _L
        lvl = pair % _L
        r = res_v[pl.ds(pair * 16, 16)]
        pltpu.sync_copy(tab_hbm.at[pl.ds(pair * _T, _T)], tab_v)
        for c in range(_CHUNKS):
            off = b * _N + c * _C
            pltpu.sync_copy(cx_hbm.at[pl.ds(off, _C)], cx_v)
            pltpu.sync_copy(cy_hbm.at[pl.ds(off, _C)], cy_v)

            def body(i, carry):
                sl = pl.ds(i * 16, 16)
                sx = cx_v[sl] * r
                sy = cy_v[sl] * r
                xi = sx.astype(jnp.int32)
                yi = sy.astype(jnp.int32)
                fx = sx - xi.astype(jnp.float32)
                fy = sy - yi.astype(jnp.float32)
                yk0 = yi * _HASH_K
                yk1 = yk0 + _HASH_K
                x1 = xi + 1
                i00 = (xi ^ yk0) & _IDX_MASK
                i10 = (x1 ^ yk0) & _IDX_MASK
                i01 = (xi ^ yk1) & _IDX_MASK
                i11 = (x1 ^ yk1) & _IDX_MASK
                g00 = plsc.load_gather(tab_v, [i00])
                g10 = plsc.load_gather(tab_v, [i10])
                g01 = plsc.load_gather(tab_v, [i01])
                g11 = plsc.load_gather(tab_v, [i11])

                def lo(g):
                    return lax.bitcast_convert_type(
                        jnp.left_shift(g, 16), jnp.float32)

                def hi(g):
                    return lax.bitcast_convert_type(
                        g & np.int32(-65536), jnp.float32)

                gx = 1.0 - fx
                gy = 1.0 - fy
                w00 = gx * gy
                w10 = fx * gy
                w01 = gx * fy
                w11 = fx * fy
                f0_v[sl] = (w00 * lo(g00) + w10 * lo(g10)
                            + w01 * lo(g01) + w11 * lo(g11))
                f1_v[sl] = (w00 * hi(g00) + w10 * hi(g10)
                            + w01 * hi(g01) + w11 * hi(g11))
                return carry

            lax.fori_loop(0, _C // 16, body, 0)
            fbase = pair * (2 * _N) + c * _C
            pltpu.sync_copy(f0_v, feats_hbm.at[pl.ds(fbase, _C)])
            pltpu.sync_copy(f1_v, feats_hbm.at[pl.ds(fbase + _N, _C)])


_sc_retrieve = functools.partial(
    pl.kernel,
    mesh=plsc.VectorSubcoreMesh(core_axis_name="c", subcore_axis_name="s"),
    out_type=jax.ShapeDtypeStruct((_PAIRS * 2 * _N,), jnp.float32),
    scratch_types=[
        pltpu.VMEM((_T,), jnp.int32),
        pltpu.VMEM((_C,), jnp.float32),
        pltpu.VMEM((_C,), jnp.float32),
        pltpu.VMEM((_C,), jnp.float32),
        pltpu.VMEM((_C,), jnp.float32),
        pltpu.VMEM((_PAIRS * 16,), jnp.float32),
    ],
    compiler_params=pltpu.CompilerParams(needs_layout_passes=False),
)(_sc_body)


_BN = 8192


def _tc_body(s_ref, awt_ref, ab_ref, w8_ref, b8_ref, feats_ref, out_ref):
    bi = pl.program_id(0)
    style = jnp.dot(s_ref[pl.ds(bi, 1), :], awt_ref[...],
                    preferred_element_type=jnp.float32) + ab_ref[...]
    w = w8_ref[...] * style
    demod = lax.rsqrt(jnp.sum(w * w, axis=1, keepdims=True) + 1e-8)
    wd = w * demod
    out_ref[0] = jnp.dot(wd, feats_ref[0],
                         preferred_element_type=jnp.float32) + b8_ref[...]


def kernel(x, coords, s, weight, bias, affine_W, affine_b):
    b = x.shape[0]
    tabs = x.reshape(b, _L, _T, 2).astype(jnp.bfloat16)
    tu = lax.bitcast_convert_type(tabs, jnp.uint16).astype(jnp.uint32)
    packed = lax.bitcast_convert_type(
        tu[..., 0] | (tu[..., 1] << 16), jnp.int32).reshape(-1)
    cx = coords[..., 0].reshape(-1)
    cy = coords[..., 1].reshape(-1)
    growth = math.exp((math.log(_RES_MAX) - math.log(_RES_MIN)) / (_L - 1))
    res = jnp.floor(_RES_MIN * growth ** jnp.arange(_L, dtype=jnp.float32))
    resx = jnp.repeat(jnp.tile(res, (_B,)), 16)

    feats = _sc_retrieve(packed, cx, cy, resx).reshape(b, 2 * _L, _N)

    w8 = jnp.zeros((8, 2 * _L), jnp.float32).at[:3].set(weight)
    b8 = jnp.zeros((8, 1), jnp.float32).at[:3, 0].set(bias)
    out_pad = pl.pallas_call(
        _tc_body,
        grid=(b, _N // _BN),
        in_specs=[
            pl.BlockSpec((_B, 512), lambda i, n: (0, 0)),
            pl.BlockSpec((512, 2 * _L), lambda i, n: (0, 0)),
            pl.BlockSpec((1, 2 * _L), lambda i, n: (0, 0)),
            pl.BlockSpec((8, 2 * _L), lambda i, n: (0, 0)),
            pl.BlockSpec((8, 1), lambda i, n: (0, 0)),
            pl.BlockSpec((1, 2 * _L, _BN), lambda i, n: (i, 0, n)),
        ],
        out_specs=pl.BlockSpec((1, 8, _BN), lambda i, n: (i, 0, n)),
        out_shape=jax.ShapeDtypeStruct((b, 8, _N), jnp.float32),
    )(s, affine_W.T, affine_b.reshape(1, -1), w8, b8, feats)

    return out_pad[:, :3, :].reshape(b, 3, _RES_MAX, _RES_MAX)


# speedup vs baseline: 575.2789x; 2.0311x over previous
import functools
import math

import jax
import jax.numpy as jnp
import numpy as np
from jax import lax
from jax.experimental import pallas as pl
from jax.experimental.pallas import tpu as pltpu
from jax.experimental.pallas import tpu_sc as plsc

_RES_MIN = 16
_RES_MAX = 256
_L = 16
_T = 65536
_B = 4
_N = _RES_MAX * _RES_MAX
_NW = 32
_PAIRS = _B * _L
_PAIRS_PER_W = _PAIRS // _NW
_C = 8192
_CHUNKS = _N // _C
_HASH_K = np.int32(-1640531535)
_IDX_MASK = np.int32(_T - 1)


_PCH = 16384


def _sc_body(x_hbm, cxy_hbm, res_hbm, feats_hbm,
             tab_v, tmp_v, cxy_v, f0_v, f1_v, res_v):
    wid = lax.axis_index("s") * 2 + lax.axis_index("c")
    pltpu.sync_copy(res_hbm, res_v)
    iota = lax.broadcasted_iota(jnp.int32, (16,), 0)

    for j in range(_PAIRS_PER_W):
        pair = wid * _PAIRS_PER_W + j
        b = pair // _L
        lvl = pair % _L
        r = res_v[pl.ds(pair * 16, 16)]

        for k in range(2 * _T // _PCH):
            pltpu.sync_copy(x_hbm.at[pl.ds(pair * 2 * _T + k * _PCH, _PCH)],
                            tmp_v)

            def pk(i, carry):
                idx = (i * 16 + iota) * 2
                ev = plsc.load_gather(tmp_v, [idx])
                od = plsc.load_gather(tmp_v, [idx + 1])
                w = plsc.bitcast(
                    plsc.pack(ev, od, format=plsc.PackFormat.INTERLEAVED),
                    jnp.int32)
                tab_v[pl.ds(k * (_PCH // 2) + i * 16, 16)] = w
                return carry

            lax.fori_loop(0, _PCH // 32, pk, 0)

        for c in range(_CHUNKS):
            pltpu.sync_copy(
                cxy_hbm.at[pl.ds(b * 2 * _N + c * 2 * _C, 2 * _C)], cxy_v)

            def body(i, carry):
                sl = pl.ds(i * 16, 16)
                idx2 = (i * 16 + iota) * 2
                sx = plsc.load_gather(cxy_v, [idx2]) * r
                sy = plsc.load_gather(cxy_v, [idx2 + 1]) * r
                xi = sx.astype(jnp.int32)
                yi = sy.astype(jnp.int32)
                fx = sx - xi.astype(jnp.float32)
                fy = sy - yi.astype(jnp.float32)
                yk0 = yi * _HASH_K
                yk1 = yk0 + _HASH_K
                x1 = xi + 1
                i00 = (xi ^ yk0) & _IDX_MASK
                i10 = (x1 ^ yk0) & _IDX_MASK
                i01 = (xi ^ yk1) & _IDX_MASK
                i11 = (x1 ^ yk1) & _IDX_MASK
                g00 = plsc.load_gather(tab_v, [i00])
                g10 = plsc.load_gather(tab_v, [i10])
                g01 = plsc.load_gather(tab_v, [i01])
                g11 = plsc.load_gather(tab_v, [i11])

                def lo(g):
                    return lax.bitcast_convert_type(
                        jnp.left_shift(g, 16), jnp.float32)

                def hi(g):
                    return lax.bitcast_convert_type(
                        g & np.int32(-65536), jnp.float32)

                gx = 1.0 - fx
                gy = 1.0 - fy
                w00 = gx * gy
                w10 = fx * gy
                w01 = gx * fy
                w11 = fx * fy
                f0_v[sl] = (w00 * lo(g00) + w10 * lo(g10)
                            + w01 * lo(g01) + w11 * lo(g11))
                f1_v[sl] = (w00 * hi(g00) + w10 * hi(g10)
                            + w01 * hi(g01) + w11 * hi(g11))
                return carry

            lax.fori_loop(0, _C // 16, body, 0)
            fbase = pair * (2 * _N) + c * _C
            pltpu.sync_copy(f0_v, feats_hbm.at[pl.ds(fbase, _C)])
            pltpu.sync_copy(f1_v, feats_hbm.at[pl.ds(fbase + _N, _C)])


_sc_retrieve = functools.partial(
    pl.kernel,
    mesh=plsc.VectorSubcoreMesh(core_axis_name="c", subcore_axis_name="s"),
    out_type=jax.ShapeDtypeStruct((_PAIRS * 2 * _N,), jnp.float32),
    scratch_types=[
        pltpu.VMEM((_T,), jnp.int32),
        pltpu.VMEM((_PCH,), jnp.float32),
        pltpu.VMEM((2 * _C,), jnp.float32),
        pltpu.VMEM((_C,), jnp.float32),
        pltpu.VMEM((_C,), jnp.float32),
        pltpu.VMEM((_PAIRS * 16,), jnp.float32),
    ],
    compiler_params=pltpu.CompilerParams(needs_layout_passes=False),
)(_sc_body)


_BN = 8192


def _tc_body(s_ref, awt_ref, ab_ref, w8_ref, b8_ref, feats_ref, out_ref):
    bi = pl.program_id(0)
    style = jnp.dot(s_ref[pl.ds(bi, 1), :], awt_ref[...],
                    preferred_element_type=jnp.float32) + ab_ref[...]
    w = w8_ref[...] * style
    demod = lax.rsqrt(jnp.sum(w * w, axis=1, keepdims=True) + 1e-8)
    wd = w * demod
    out_ref[0] = jnp.dot(wd, feats_ref[0],
                         preferred_element_type=jnp.float32) + b8_ref[...]


def kernel(x, coords, s, weight, bias, affine_W, affine_b):
    b = x.shape[0]
    growth = math.exp((math.log(_RES_MAX) - math.log(_RES_MIN)) / (_L - 1))
    res = jnp.floor(_RES_MIN * growth ** jnp.arange(_L, dtype=jnp.float32))
    resx = jnp.repeat(jnp.tile(res, (_B,)), 16)

    feats = _sc_retrieve(x.reshape(-1), coords.reshape(-1),
                         resx).reshape(b, 2 * _L, _N)

    w8 = jnp.zeros((8, 2 * _L), jnp.float32).at[:3].set(weight)
    b8 = jnp.zeros((8, 1), jnp.float32).at[:3, 0].set(bias)
    out_pad = pl.pallas_call(
        _tc_body,
        grid=(b, _N // _BN),
        in_specs=[
            pl.BlockSpec((_B, 512), lambda i, n: (0, 0)),
            pl.BlockSpec((512, 2 * _L), lambda i, n: (0, 0)),
            pl.BlockSpec((1, 2 * _L), lambda i, n: (0, 0)),
            pl.BlockSpec((8, 2 * _L), lambda i, n: (0, 0)),
            pl.BlockSpec((8, 1), lambda i, n: (0, 0)),
            pl.BlockSpec((1, 2 * _L, _BN), lambda i, n: (i, 0, n)),
        ],
        out_specs=pl.BlockSpec((1, 8, _BN), lambda i, n: (i, 0, n)),
        out_shape=jax.ShapeDtypeStruct((b, 8, _N), jnp.float32),
    )(s, affine_W.T, affine_b.reshape(1, -1), w8, b8, feats)

    return out_pad[:, :3, :].reshape(b, 3, _RES_MAX, _RES_MAX)
